# baseline (speedup 1.0000x reference)
import functools

import jax
import jax.numpy as jnp
import numpy as np
from jax import lax
from jax.experimental import pallas as pl
from jax.experimental.pallas import tpu as pltpu
from jax.experimental.pallas import tpu_sc as plsc

N = 10000
E = 320000
D = 128

NC = 2
NS = 16
NW = NC * NS
CH = 80
CPT = E // (NW * CH)
NPAD = 10240
RPT = NPAD // NS

BM = 1000


def _mm_aug(x, w):
    def body(x_ref, w_ref, o_ref):
        y = lax.dot_general(x_ref[...], w_ref[...], (((1,), (1,)), ((), ())),
                            preferred_element_type=jnp.float32)
        ones = jnp.ones((BM, 1), jnp.float32)
        zer = jnp.zeros((BM, 15), jnp.float32)
        o_ref[...] = jnp.concatenate([y, ones, zer], axis=1)

    return pl.pallas_call(
        body,
        grid=(N // BM,),
        in_specs=[pl.BlockSpec((BM, D), lambda i: (i, 0)),
                  pl.BlockSpec((D, D), lambda i: (0, 0))],
        out_specs=pl.BlockSpec((BM, D + 16), lambda i: (i, 0)),
        out_shape=jax.ShapeDtypeStruct((N, D + 16), jnp.float32),
    )(x, w)


def _combine1(a0, a1, x, wr, bl, wl2):
    def body(a0_ref, a1_ref, x_ref, wr_ref, bl_ref, wl2_ref,
             h_ref, y2_ref, inv_ref):
        s = a0_ref[0] + a1_ref[0]
        inv = 1.0 / jnp.maximum(s[:, D:D + 1], 1.0)
        lin = lax.dot_general(x_ref[...], wr_ref[...], (((1,), (1,)), ((), ())),
                              preferred_element_type=jnp.float32)
        h = jnp.maximum(s[:, :D] * inv + bl_ref[...] + lin, 0.0)
        h_ref[...] = h
        y2_ref[...] = lax.dot_general(h, wl2_ref[...], (((1,), (1,)), ((), ())),
                                      preferred_element_type=jnp.float32)
        inv_ref[...] = inv

    return pl.pallas_call(
        body,
        grid=(N // BM,),
        in_specs=[pl.BlockSpec((1, BM, D + 16), lambda i: (0, i, 0)),
                  pl.BlockSpec((1, BM, D + 16), lambda i: (1, i, 0)),
                  pl.BlockSpec((BM, D), lambda i: (i, 0)),
                  pl.BlockSpec((D, D), lambda i: (0, 0)),
                  pl.BlockSpec((1, D), lambda i: (0, 0)),
                  pl.BlockSpec((D, D), lambda i: (0, 0))],
        out_specs=[pl.BlockSpec((BM, D), lambda i: (i, 0)),
                   pl.BlockSpec((BM, D), lambda i: (i, 0)),
                   pl.BlockSpec((BM, 1), lambda i: (i, 0))],
        out_shape=[jax.ShapeDtypeStruct((N, D), jnp.float32),
                   jax.ShapeDtypeStruct((N, D), jnp.float32),
                   jax.ShapeDtypeStruct((N, 1), jnp.float32)],
    )(a0, a1, x, wr, bl, wl2)


def _combine2(a0, a1, h, wr, bl, inv):
    def body(a0_ref, a1_ref, h_ref, wr_ref, bl_ref, inv_ref, zbf_ref):
        s = a0_ref[0] + a1_ref[0]
        lin = lax.dot_general(h_ref[...], wr_ref[...], (((1,), (1,)), ((), ())),
                              preferred_element_type=jnp.float32)
        z = s * inv_ref[...] + bl_ref[...] + lin
        zbf_ref[...] = z.astype(jnp.bfloat16)

    return pl.pallas_call(
        body,
        grid=(N // BM,),
        in_specs=[pl.BlockSpec((1, BM, D), lambda i: (0, i, 0)),
                  pl.BlockSpec((1, BM, D), lambda i: (1, i, 0)),
                  pl.BlockSpec((BM, D), lambda i: (i, 0)),
                  pl.BlockSpec((D, D), lambda i: (0, 0)),
                  pl.BlockSpec((1, D), lambda i: (0, 0)),
                  pl.BlockSpec((BM, 1), lambda i: (i, 0))],
        out_specs=pl.BlockSpec((BM, D), lambda i: (i, 0)),
        out_shape=jax.ShapeDtypeStruct((N, D), jnp.bfloat16),
    )(a0, a1, h, wr, bl, inv)


def _make_agg(width, ch):
    mesh = plsc.VectorSubcoreMesh(core_axis_name="c", subcore_axis_name="s")
    cpt = E // (NW * ch)

    def body(y, src2d, dst2d, zeros, out, ia, ib, rows, accum,
             semi, semg, sems):
        cid = lax.axis_index("c")
        sid = lax.axis_index("s")
        wid = sid * NC + cid
        base = wid * cpt
        pltpu.sync_copy(zeros.at[pl.ds(sid * RPT, RPT)],
                        accum.at[pl.ds(sid * RPT, RPT)])
        plsc.subcore_barrier()

        def issue_i(j, b6):
            @pl.when(j < cpt)
            def _():
                pltpu.async_copy(src2d.at[base + j], ia.at[b6], semi.at[2 * b6])
                pltpu.async_copy(dst2d.at[base + j], ib.at[b6], semi.at[2 * b6 + 1])

        def wait_i(j, b6):
            pltpu.make_async_copy(src2d.at[base + j], ia.at[b6],
                                  semi.at[2 * b6]).wait()
            pltpu.make_async_copy(dst2d.at[base + j], ib.at[b6],
                                  semi.at[2 * b6 + 1]).wait()

        def issue_g(j, b3, b6):
            pltpu.async_copy(y.at[ia.at[b6]], rows.at[b3], semg.at[b3])

        def wait_g(j, b3, b6):
            pltpu.make_async_copy(y.at[ia.at[b6]], rows.at[b3],
                                  semg.at[b3]).wait()

        def issue_s(j, b3, b6):
            pltpu.async_copy(rows.at[b3], accum.at[ib.at[b6]],
                             sems.at[b3], add=True)

        def wait_s(j, b3, b6):
            pltpu.make_async_copy(rows.at[b3], accum.at[ib.at[b6]],
                                  sems.at[b3]).wait()

        for k in range(4):
            issue_i(k, k)
        wait_i(0, 0)
        issue_g(0, 0, 0)
        wait_i(1, 1)
        issue_g(1, 1, 1)

        def six(t, carry):
            j0 = 6 * t
            for s in range(6):
                j = j0 + s

                @pl.when(j < cpt)
                def _():
                    wait_g(j, s % 3, s)
                    issue_s(j, s % 3, s)

                    @pl.when(j + 2 < cpt)
                    def _():
                        @pl.when(j >= 1)
                        def _():
                            wait_s(j - 1, (s - 1) % 3, (s - 1) % 6)

                        wait_i(j + 2, (s + 2) % 6)
                        issue_g(j + 2, (s + 2) % 3, (s + 2) % 6)

                    issue_i(j + 4, (s + 4) % 6)
            return carry

        lax.fori_loop(0, (cpt + 5) // 6, six, 0)
        wait_s(cpt - 3, (cpt - 3) % 3, (cpt - 3) % 6)
        wait_s(cpt - 2, (cpt - 2) % 3, (cpt - 2) % 6)
        wait_s(cpt - 1, (cpt - 1) % 3, (cpt - 1) % 6)
        plsc.subcore_barrier()
        pltpu.sync_copy(accum.at[pl.ds(sid * RPT, RPT)],
                        out.at[cid, pl.ds(sid * RPT, RPT)])

    return pl.kernel(
        body,
        out_type=jax.ShapeDtypeStruct((NC, NPAD, width), jnp.float32),
        mesh=mesh,
        compiler_params=pltpu.CompilerParams(use_tc_tiling_on_sc=False),
        scratch_types=[
            pltpu.VMEM((6, ch), jnp.int32),
            pltpu.VMEM((6, ch), jnp.int32),
            pltpu.VMEM((3, ch, width), jnp.float32),
            pltpu.VMEM_SHARED((NPAD, width), jnp.float32),
            pltpu.SemaphoreType.DMA((12,)),
            pltpu.SemaphoreType.DMA((3,)),
            pltpu.SemaphoreType.DMA((3,)),
        ],
    )


def _make_decode():
    mesh = plsc.VectorSubcoreMesh(core_axis_name="c", subcore_axis_name="s")

    def body(z, s2d, d2d, out, ia_all, ib_all, av, bv, ov, mbuf, zsh, sems):
        cid = lax.axis_index("c")
        sid = lax.axis_index("s")
        wid = sid * NC + cid
        base = wid * CPT
        pltpu.sync_copy(s2d.at[pl.ds(base, CPT)], ia_all)
        pltpu.sync_copy(d2d.at[pl.ds(base, CPT)], ib_all)

        @pl.when(sid < 10)
        def _():
            pltpu.sync_copy(z.at[pl.ds(sid * 1000, 1000)],
                            zsh.at[pl.ds(sid * 1000, 1000)])

        plsc.subcore_barrier()

        def issue(j, buf):
            pltpu.async_copy(zsh.at[ia_all.at[j]], av.at[buf], sems.at[2 * buf])
            pltpu.async_copy(zsh.at[ib_all.at[j]], bv.at[buf], sems.at[2 * buf + 1])

        def wait(j, buf):
            pltpu.make_async_copy(zsh.at[ia_all.at[j]], av.at[buf],
                                  sems.at[2 * buf]).wait()
            pltpu.make_async_copy(zsh.at[ib_all.at[j]], bv.at[buf],
                                  sems.at[2 * buf + 1]).wait()

        lanes = lax.iota(jnp.int32, 16)

        def compute(j, buf):
            for g in range(CH // 16):
                for e in range(16):
                    ee = g * 16 + e
                    pms = [av[buf, ee, pl.ds(q * 32, 32)]
                           * bv[buf, ee, pl.ds(q * 32, 32)]
                           for q in range(D // 32)]
                    s01 = pms[0] + pms[1]
                    s23 = pms[2] + pms[3]
                    m1, m2 = plsc.unpack(s01, format=plsc.PackFormat.INTERLEAVED)
                    m3, m4 = plsc.unpack(s23, format=plsc.PackFormat.INTERLEAVED)
                    mbuf[pl.ds(e * 16, 16)] = (m1 + m2) + (m3 + m4)
                acc0 = plsc.load_gather(mbuf, [lanes * 16])
                acc1 = plsc.load_gather(mbuf, [lanes * 16 + 1])
                for c in range(2, 16, 2):
                    acc0 = acc0 + plsc.load_gather(mbuf, [lanes * 16 + c])
                    acc1 = acc1 + plsc.load_gather(mbuf, [lanes * 16 + c + 1])
                ov[pl.ds(g * 16, 16)] = acc0 + acc1
            pltpu.sync_copy(ov, out.at[pl.ds((base + j) * CH, CH)])

        issue(0, 0)
        issue(1, 1)
        issue(2, 2)

        def triple(t, carry):
            j0 = 3 * t
            for s in range(3):
                j = j0 + s

                @pl.when(j < CPT)
                def _():
                    wait(j, s)
                    compute(j, s)

                    @pl.when(j + 3 < CPT)
                    def _():
                        issue(j + 3, s)
            return carry

        lax.fori_loop(0, (CPT + 2) // 3, triple, 0)

    return pl.kernel(
        body,
        out_type=jax.ShapeDtypeStruct((E,), jnp.float32),
        mesh=mesh,
        compiler_params=pltpu.CompilerParams(use_tc_tiling_on_sc=False,
                                             needs_layout_passes=False),
        scratch_types=[
            pltpu.VMEM((CPT, CH), jnp.int32),
            pltpu.VMEM((CPT, CH), jnp.int32),
            pltpu.VMEM((3, CH, D), jnp.bfloat16),
            pltpu.VMEM((3, CH, D), jnp.bfloat16),
            pltpu.VMEM((CH,), jnp.float32),
            pltpu.VMEM((256,), jnp.float32),
            pltpu.VMEM_SHARED((N, D), jnp.bfloat16),
            pltpu.SemaphoreType.DMA((6,)),
        ],
    )


CH1 = 40
_agg_aug = _make_agg(D + 16, CH1)
_agg_plain = _make_agg(D, CH)
_decode = _make_decode()


def kernel(x, edge_index, edge_label_index, Wl1, bl1, Wr1, Wl2, bl2, Wr2):
    src1 = edge_index[0].astype(jnp.int32).reshape(E // CH1, CH1)
    dst1 = edge_index[1].astype(jnp.int32).reshape(E // CH1, CH1)
    src2 = edge_index[0].astype(jnp.int32).reshape(E // CH, CH)
    dst2 = edge_index[1].astype(jnp.int32).reshape(E // CH, CH)
    es2d = edge_label_index[0].astype(jnp.int32).reshape(E // CH, CH)
    ed2d = edge_label_index[1].astype(jnp.int32).reshape(E // CH, CH)
    zeros_aug = jnp.zeros((NPAD, D + 16), jnp.float32)
    zeros_pln = jnp.zeros((NPAD, D), jnp.float32)

    y1 = _mm_aug(x, Wl1)
    agg1 = _agg_aug(y1, src1, dst1, zeros_aug)
    h, y2, inv = _combine1(agg1, agg1, x, Wr1, bl1.reshape(1, D), Wl2)

    agg2 = _agg_plain(y2, src2, dst2, zeros_pln)
    z = _combine2(agg2, agg2, h, Wr2, bl2.reshape(1, D), inv)

    return _decode(z, es2d, ed2d)


# speedup vs baseline: 1.0370x; 1.0370x over previous
import functools

import jax
import jax.numpy as jnp
import numpy as np
from jax import lax
from jax.experimental import pallas as pl
from jax.experimental.pallas import tpu as pltpu
from jax.experimental.pallas import tpu_sc as plsc

N = 10000
E = 320000
D = 128

NC = 2
NS = 16
NW = NC * NS
CH = 80
CPT = E // (NW * CH)
NPAD = 10240
RPT = NPAD // NS

BM = 1000


def _mm_aug(x, w):
    def body(x_ref, w_ref, o_ref):
        y = lax.dot_general(x_ref[...], w_ref[...], (((1,), (1,)), ((), ())),
                            preferred_element_type=jnp.float32)
        ones = jnp.ones((BM, 1), jnp.float32)
        zer = jnp.zeros((BM, 15), jnp.float32)
        o_ref[...] = jnp.concatenate([y, ones, zer], axis=1)

    return pl.pallas_call(
        body,
        grid=(N // BM,),
        in_specs=[pl.BlockSpec((BM, D), lambda i: (i, 0)),
                  pl.BlockSpec((D, D), lambda i: (0, 0))],
        out_specs=pl.BlockSpec((BM, D + 16), lambda i: (i, 0)),
        out_shape=jax.ShapeDtypeStruct((N, D + 16), jnp.float32),
    )(x, w)


def _combine1(a0, a1, x, wr, bl, wl2):
    def body(a0_ref, a1_ref, x_ref, wr_ref, bl_ref, wl2_ref,
             h_ref, y2_ref, inv_ref):
        s = a0_ref[0] + a1_ref[0]
        inv = 1.0 / jnp.maximum(s[:, D:D + 1], 1.0)
        lin = lax.dot_general(x_ref[...], wr_ref[...], (((1,), (1,)), ((), ())),
                              preferred_element_type=jnp.float32)
        h = jnp.maximum(s[:, :D] * inv + bl_ref[...] + lin, 0.0)
        h_ref[...] = h
        y2_ref[...] = lax.dot_general(h, wl2_ref[...], (((1,), (1,)), ((), ())),
                                      preferred_element_type=jnp.float32)
        inv_ref[...] = inv

    return pl.pallas_call(
        body,
        grid=(N // BM,),
        in_specs=[pl.BlockSpec((1, BM, D + 16), lambda i: (0, i, 0)),
                  pl.BlockSpec((1, BM, D + 16), lambda i: (1, i, 0)),
                  pl.BlockSpec((BM, D), lambda i: (i, 0)),
                  pl.BlockSpec((D, D), lambda i: (0, 0)),
                  pl.BlockSpec((1, D), lambda i: (0, 0)),
                  pl.BlockSpec((D, D), lambda i: (0, 0))],
        out_specs=[pl.BlockSpec((BM, D), lambda i: (i, 0)),
                   pl.BlockSpec((BM, D), lambda i: (i, 0)),
                   pl.BlockSpec((BM, 1), lambda i: (i, 0))],
        out_shape=[jax.ShapeDtypeStruct((N, D), jnp.float32),
                   jax.ShapeDtypeStruct((N, D), jnp.float32),
                   jax.ShapeDtypeStruct((N, 1), jnp.float32)],
    )(a0, a1, x, wr, bl, wl2)


def _combine2(a0, a1, h, wr, bl, inv):
    def body(a0_ref, a1_ref, h_ref, wr_ref, bl_ref, inv_ref, zbf_ref):
        s = a0_ref[0] + a1_ref[0]
        lin = lax.dot_general(h_ref[...], wr_ref[...], (((1,), (1,)), ((), ())),
                              preferred_element_type=jnp.float32)
        z = s * inv_ref[...] + bl_ref[...] + lin
        zbf_ref[...] = z.astype(jnp.bfloat16)

    return pl.pallas_call(
        body,
        grid=(N // BM,),
        in_specs=[pl.BlockSpec((1, BM, D), lambda i: (0, i, 0)),
                  pl.BlockSpec((1, BM, D), lambda i: (1, i, 0)),
                  pl.BlockSpec((BM, D), lambda i: (i, 0)),
                  pl.BlockSpec((D, D), lambda i: (0, 0)),
                  pl.BlockSpec((1, D), lambda i: (0, 0)),
                  pl.BlockSpec((BM, 1), lambda i: (i, 0))],
        out_specs=pl.BlockSpec((BM, D), lambda i: (i, 0)),
        out_shape=jax.ShapeDtypeStruct((N, D), jnp.bfloat16),
    )(a0, a1, h, wr, bl, inv)


def _make_agg(width, ch):
    mesh = plsc.VectorSubcoreMesh(core_axis_name="c", subcore_axis_name="s")
    cpt = E // (NW * ch)

    def body(y, src2d, dst2d, zeros, out, ia, ib, rows, accum,
             semi, semg, sems):
        cid = lax.axis_index("c")
        sid = lax.axis_index("s")
        wid = sid * NC + cid
        base = wid * cpt
        pltpu.sync_copy(zeros.at[pl.ds(sid * RPT, RPT)],
                        accum.at[pl.ds(sid * RPT, RPT)])
        plsc.subcore_barrier()

        def issue_i(j, b6):
            @pl.when(j < cpt)
            def _():
                pltpu.async_copy(src2d.at[base + j], ia.at[b6], semi.at[2 * b6])
                pltpu.async_copy(dst2d.at[base + j], ib.at[b6], semi.at[2 * b6 + 1])

        def wait_i(j, b6):
            pltpu.make_async_copy(src2d.at[base + j], ia.at[b6],
                                  semi.at[2 * b6]).wait()
            pltpu.make_async_copy(dst2d.at[base + j], ib.at[b6],
                                  semi.at[2 * b6 + 1]).wait()

        def issue_g(j, b3, b6):
            pltpu.async_copy(y.at[ia.at[b6]], rows.at[b3], semg.at[b3])

        def wait_g(j, b3, b6):
            pltpu.make_async_copy(y.at[ia.at[b6]], rows.at[b3],
                                  semg.at[b3]).wait()

        def issue_s(j, b3, b6):
            pltpu.async_copy(rows.at[b3], accum.at[ib.at[b6]],
                             sems.at[b3], add=True)

        def wait_s(j, b3, b6):
            pltpu.make_async_copy(rows.at[b3], accum.at[ib.at[b6]],
                                  sems.at[b3]).wait()

        for k in range(4):
            issue_i(k, k)
        wait_i(0, 0)
        issue_g(0, 0, 0)
        wait_i(1, 1)
        issue_g(1, 1, 1)

        def six(t, carry):
            j0 = 6 * t
            for s in range(6):
                j = j0 + s

                @pl.when(j < cpt)
                def _():
                    wait_g(j, s % 3, s)
                    issue_s(j, s % 3, s)

                    @pl.when(j + 2 < cpt)
                    def _():
                        @pl.when(j >= 1)
                        def _():
                            wait_s(j - 1, (s - 1) % 3, (s - 1) % 6)

                        wait_i(j + 2, (s + 2) % 6)
                        issue_g(j + 2, (s + 2) % 3, (s + 2) % 6)

                    issue_i(j + 4, (s + 4) % 6)
            return carry

        lax.fori_loop(0, (cpt + 5) // 6, six, 0)
        wait_s(cpt - 3, (cpt - 3) % 3, (cpt - 3) % 6)
        wait_s(cpt - 2, (cpt - 2) % 3, (cpt - 2) % 6)
        wait_s(cpt - 1, (cpt - 1) % 3, (cpt - 1) % 6)
        plsc.subcore_barrier()
        pltpu.sync_copy(accum.at[pl.ds(sid * RPT, RPT)],
                        out.at[cid, pl.ds(sid * RPT, RPT)])

    return pl.kernel(
        body,
        out_type=jax.ShapeDtypeStruct((NC, NPAD, width), jnp.float32),
        mesh=mesh,
        compiler_params=pltpu.CompilerParams(use_tc_tiling_on_sc=False),
        scratch_types=[
            pltpu.VMEM((6, ch), jnp.int32),
            pltpu.VMEM((6, ch), jnp.int32),
            pltpu.VMEM((3, ch, width), jnp.float32),
            pltpu.VMEM_SHARED((NPAD, width), jnp.float32),
            pltpu.SemaphoreType.DMA((12,)),
            pltpu.SemaphoreType.DMA((3,)),
            pltpu.SemaphoreType.DMA((3,)),
        ],
    )


def _make_decode():
    mesh = plsc.VectorSubcoreMesh(core_axis_name="c", subcore_axis_name="s")

    def body(z, s2d, d2d, out, ia_all, ib_all, av, bv, ov, mbuf, zsh, sems):
        cid = lax.axis_index("c")
        sid = lax.axis_index("s")
        wid = sid * NC + cid
        base = wid * CPT
        pltpu.sync_copy(s2d.at[pl.ds(base, CPT)], ia_all)
        pltpu.sync_copy(d2d.at[pl.ds(base, CPT)], ib_all)

        @pl.when(sid < 10)
        def _():
            pltpu.sync_copy(z.at[pl.ds(sid * 1000, 1000)],
                            zsh.at[pl.ds(sid * 1000, 1000)])

        plsc.subcore_barrier()

        def issue(j, buf):
            pltpu.async_copy(zsh.at[ia_all.at[j]], av.at[buf], sems.at[2 * buf])
            pltpu.async_copy(zsh.at[ib_all.at[j]], bv.at[buf], sems.at[2 * buf + 1])

        def wait(j, buf):
            pltpu.make_async_copy(zsh.at[ia_all.at[j]], av.at[buf],
                                  sems.at[2 * buf]).wait()
            pltpu.make_async_copy(zsh.at[ib_all.at[j]], bv.at[buf],
                                  sems.at[2 * buf + 1]).wait()

        lanes = lax.iota(jnp.int32, 16)

        def compute(j, buf):
            for g in range(CH // 16):
                for e in range(16):
                    ee = g * 16 + e
                    p0 = jnp.zeros((16,), jnp.float32)
                    p1 = jnp.zeros((16,), jnp.float32)
                    for q in range(D // 32):
                        pm = (av[buf, ee, pl.ds(q * 32, 32)]
                              * bv[buf, ee, pl.ds(q * 32, 32)])
                        m1, m2 = plsc.unpack(pm, format=plsc.PackFormat.INTERLEAVED)
                        p0 = p0 + m1
                        p1 = p1 + m2
                    mbuf[pl.ds(e * 16, 16)] = p0 + p1
                acc0 = plsc.load_gather(mbuf, [lanes * 16])
                acc1 = plsc.load_gather(mbuf, [lanes * 16 + 1])
                for c in range(2, 16, 2):
                    acc0 = acc0 + plsc.load_gather(mbuf, [lanes * 16 + c])
                    acc1 = acc1 + plsc.load_gather(mbuf, [lanes * 16 + c + 1])
                ov[pl.ds(g * 16, 16)] = acc0 + acc1
            pltpu.sync_copy(ov, out.at[pl.ds((base + j) * CH, CH)])

        issue(0, 0)
        issue(1, 1)
        issue(2, 2)

        def triple(t, carry):
            j0 = 3 * t
            for s in range(3):
                j = j0 + s

                @pl.when(j < CPT)
                def _():
                    wait(j, s)
                    compute(j, s)

                    @pl.when(j + 3 < CPT)
                    def _():
                        issue(j + 3, s)
            return carry

        lax.fori_loop(0, (CPT + 2) // 3, triple, 0)

    return pl.kernel(
        body,
        out_type=jax.ShapeDtypeStruct((E,), jnp.float32),
        mesh=mesh,
        compiler_params=pltpu.CompilerParams(use_tc_tiling_on_sc=False,
                                             needs_layout_passes=False),
        scratch_types=[
            pltpu.VMEM((CPT, CH), jnp.int32),
            pltpu.VMEM((CPT, CH), jnp.int32),
            pltpu.VMEM((3, CH, D), jnp.bfloat16),
            pltpu.VMEM((3, CH, D), jnp.bfloat16),
            pltpu.VMEM((CH,), jnp.float32),
            pltpu.VMEM((256,), jnp.float32),
            pltpu.VMEM_SHARED((N, D), jnp.bfloat16),
            pltpu.SemaphoreType.DMA((6,)),
        ],
    )


CH1 = 40
_agg_aug = _make_agg(D + 16, CH1)
_agg_plain = _make_agg(D, CH)
_decode = _make_decode()


def kernel(x, edge_index, edge_label_index, Wl1, bl1, Wr1, Wl2, bl2, Wr2):
    src1 = edge_index[0].astype(jnp.int32).reshape(E // CH1, CH1)
    dst1 = edge_index[1].astype(jnp.int32).reshape(E // CH1, CH1)
    src2 = edge_index[0].astype(jnp.int32).reshape(E // CH, CH)
    dst2 = edge_index[1].astype(jnp.int32).reshape(E // CH, CH)
    es2d = edge_label_index[0].astype(jnp.int32).reshape(E // CH, CH)
    ed2d = edge_label_index[1].astype(jnp.int32).reshape(E // CH, CH)
    zeros_aug = jnp.zeros((NPAD, D + 16), jnp.float32)
    zeros_pln = jnp.zeros((NPAD, D), jnp.float32)

    y1 = _mm_aug(x, Wl1)
    agg1 = _agg_aug(y1, src1, dst1, zeros_aug)
    h, y2, inv = _combine1(agg1, agg1, x, Wr1, bl1.reshape(1, D), Wl2)

    agg2 = _agg_plain(y2, src2, dst2, zeros_pln)
    z = _combine2(agg2, agg2, h, Wr2, bl2.reshape(1, D), inv)

    return _decode(z, es2d, ed2d)
